# baseline (speedup 1.0000x reference)
import functools

import jax
import jax.numpy as jnp
from jax import lax
from jax.experimental import pallas as pl
from jax.experimental.pallas import tpu as pltpu
from jax.experimental.pallas import tpu_sc as plsc

_N_ROWS = 65536
_W = 512
_N_SLABS = 128
_SC_SLABS = 64
_N_WORKERS = 32
_SC_ROWS = _SC_SLABS * 512
_ROWS_PER_W = _SC_ROWS // _N_WORKERS
_CHUNK = 16
_N_CHUNKS = _ROWS_PER_W // _CHUNK
_NBUF = 8
_OUT_ROWS_PER_W = _ROWS_PER_W * 16 // 128
_MAGIC = 12582912.0


def _sc_kernel(x_hbm, g_hbm, out_hbm, g_v, stage0_v, stage1_v, stage2_v,
               stage3_v, stage4_v, stage5_v, stage6_v, stage7_v, out_v, sems):
    stage = (stage0_v, stage1_v, stage2_v, stage3_v, stage4_v, stage5_v,
             stage6_v, stage7_v)
    wid = lax.axis_index("s") * 2 + lax.axis_index("c")
    row_base = wid * _ROWS_PER_W

    pltpu.sync_copy(g_hbm, g_v)
    g = g_v[...]
    t = (g + 1.0) * ((_W - 1) / 2.0)
    r = (t + _MAGIC) - _MAGIC
    idx = r.astype(jnp.int32)
    idx = jnp.minimum(jnp.maximum(idx, 0), _W - 1)

    k_iota = lax.iota(jnp.int32, 16)
    row_off = (k_iota >> 1) * 8 + (k_iota & 1)

    def start_fetch(c, b):
        pltpu.async_copy(
            x_hbm.at[pl.ds(row_base + c * _CHUNK, _CHUNK), :],
            stage[b],
            sems.at[b],
        )

    def wait_fetch(c, b):
        pltpu.make_async_copy(
            x_hbm.at[pl.ds(row_base + c * _CHUNK, _CHUNK), :],
            stage[b],
            sems.at[b],
        ).wait()

    for b in range(_NBUF):
        start_fetch(b, b)

    @pl.loop(0, _N_CHUNKS, step=_NBUF)
    def chunk_group(c0):
        for b in range(_NBUF):
            c = c0 + b
            wait_fetch(c, b)
            cps = 512 // _CHUNK
            slab = c // cps
            r_chunk = (c % cps) * _CHUNK

            def row_body(j, _):
                rows = jax.lax.broadcast(j, (16,))
                vals = plsc.load_gather(stage[b], [rows, idx])
                rr = r_chunk + j
                out_row0 = slab * 64 + (rr >> 7) * 2
                out_rows = jax.lax.broadcast(out_row0, (16,)) + row_off
                out_lanes = jax.lax.broadcast(rr & 127, (16,))
                plsc.store_scatter(out_v, [out_rows, out_lanes], vals)
                return _

            lax.fori_loop(0, _CHUNK, row_body, None, unroll=8)

            @pl.when(c + _NBUF < _N_CHUNKS)
            def _():
                start_fetch(c + _NBUF, b)

    pltpu.sync_copy(
        out_v, out_hbm.at[pl.ds(wid * _OUT_ROWS_PER_W, _OUT_ROWS_PER_W), :]
    )


def _sc_gather(x2, gflat):
    mesh = plsc.VectorSubcoreMesh(core_axis_name="c", subcore_axis_name="s")
    run = functools.partial(
        pl.kernel,
        mesh=mesh,
        compiler_params=pltpu.CompilerParams(needs_layout_passes=False),
        out_type=jax.ShapeDtypeStruct((_SC_ROWS * 16 // 128, 128), jnp.float32),
        scratch_types=[
            pltpu.VMEM((16,), jnp.float32),
            pltpu.VMEM((_CHUNK, _W), jnp.float32),
            pltpu.VMEM((_CHUNK, _W), jnp.float32),
            pltpu.VMEM((_CHUNK, _W), jnp.float32),
            pltpu.VMEM((_CHUNK, _W), jnp.float32),
            pltpu.VMEM((_CHUNK, _W), jnp.float32),
            pltpu.VMEM((_CHUNK, _W), jnp.float32),
            pltpu.VMEM((_CHUNK, _W), jnp.float32),
            pltpu.VMEM((_CHUNK, _W), jnp.float32),
            pltpu.VMEM((_OUT_ROWS_PER_W, 128), jnp.float32),
            pltpu.SemaphoreType.DMA((_NBUF,)),
        ],
    )(_sc_kernel)
    return run(x2, gflat)


def _tc_kernel(grid_ref, x_ref, out_ref):
    g = grid_ref[0:1, :]
    gi = jnp.round((g + 1.0) * (_W - 1) / 2.0).astype(jnp.int32)
    gi = jnp.clip(gi, 0, _W - 1)
    lane = jax.lax.broadcasted_iota(jnp.int32, (_W, 16), 0)
    onehot = (lane == jnp.broadcast_to(gi, (_W, 16))).astype(jnp.float32)
    yt = jax.lax.dot_general(
        onehot, x_ref[...], (((0,), (1,)), ((), ())),
        preferred_element_type=jnp.float32,
        precision=jax.lax.Precision.HIGHEST,
    )
    for rt in range(4):
        for k in range(16):
            row = (k >> 1) * 8 + rt * 2 + (k & 1)
            out_ref[row:row + 1, :] = yt[k:k + 1, rt * 128:(rt + 1) * 128]


def _tc_gather(x2, grid):
    n_tc_slabs = _N_SLABS - _SC_SLABS
    g2d = jnp.broadcast_to(grid.reshape(1, -1), (8, grid.size))
    return pl.pallas_call(
        _tc_kernel,
        grid=(n_tc_slabs,),
        in_specs=[
            pl.BlockSpec((8, 16), lambda s: (0, 0)),
            pl.BlockSpec((512, _W), lambda s: (_SC_SLABS + s, 0)),
        ],
        out_specs=pl.BlockSpec((64, 128), lambda s: (s, 0)),
        out_shape=jax.ShapeDtypeStruct((n_tc_slabs * 64, 128), jnp.float32),
    )(g2d, x2)


def kernel(x, grid):
    b, c, r, w = x.shape
    x2 = x.reshape(b * c * r, w)
    gflat = grid.reshape(grid.size)

    sc_out = _sc_gather(x2, gflat)
    tc_out = _tc_gather(x2, grid)
    out = jnp.concatenate([sc_out, tc_out], axis=0)

    out6 = (
        out.reshape(b, c, 8, 4, 2, 128)
        .transpose(0, 1, 3, 5, 2, 4)
        .reshape(b, c, r, 8, 1, 2)
    )
    return out6


# speedup vs baseline: 1.4548x; 1.4548x over previous
import functools

import jax
import jax.numpy as jnp
from jax import lax
from jax.experimental import pallas as pl
from jax.experimental.pallas import tpu as pltpu
from jax.experimental.pallas import tpu_sc as plsc

_N_ROWS = 65536
_W = 512
_N_SLABS = 128
_SC_SLABS = 96
_N_WORKERS = 32
_SC_ROWS = _SC_SLABS * 512
_ROWS_PER_W = _SC_ROWS // _N_WORKERS
_CHUNK = 8
_N_CHUNKS = _ROWS_PER_W // _CHUNK
_NBUF = 16
_OUT_ROWS_PER_W = _ROWS_PER_W * 16 // 128
_MAGIC = 12582912.0


def _sc_kernel(x_hbm, g_hbm, out_hbm, g_v, stage0_v, stage1_v, stage2_v,
               stage3_v, stage4_v, stage5_v, stage6_v, stage7_v, stage8_v,
               stage9_v, stage10_v, stage11_v, stage12_v, stage13_v,
               stage14_v, stage15_v, out_v, sems):
    stage = (stage0_v, stage1_v, stage2_v, stage3_v, stage4_v, stage5_v,
             stage6_v, stage7_v, stage8_v, stage9_v, stage10_v, stage11_v,
             stage12_v, stage13_v, stage14_v, stage15_v)
    wid = lax.axis_index("s") * 2 + lax.axis_index("c")
    row_base = wid * _ROWS_PER_W

    pltpu.sync_copy(g_hbm, g_v)
    g = g_v[...]
    t = (g + 1.0) * ((_W - 1) / 2.0)
    r = (t + _MAGIC) - _MAGIC
    idx = r.astype(jnp.int32)
    idx = jnp.minimum(jnp.maximum(idx, 0), _W - 1)

    k_iota = lax.iota(jnp.int32, 16)
    row_off = (k_iota >> 1) * 8 + (k_iota & 1)

    def start_fetch(c, b):
        pltpu.async_copy(
            x_hbm.at[pl.ds(row_base + c * _CHUNK, _CHUNK), :],
            stage[b],
            sems.at[b],
        )

    def wait_fetch(c, b):
        pltpu.make_async_copy(
            x_hbm.at[pl.ds(row_base + c * _CHUNK, _CHUNK), :],
            stage[b],
            sems.at[b],
        ).wait()

    for b in range(_NBUF):
        start_fetch(b, b)

    @pl.loop(0, _N_CHUNKS, step=_NBUF)
    def chunk_group(c0):
        for b in range(_NBUF):
            c = c0 + b
            wait_fetch(c, b)
            cps = 512 // _CHUNK
            slab = c // cps
            r_chunk = (c % cps) * _CHUNK

            def row_body(j, _):
                rows = jax.lax.broadcast(j, (16,))
                vals = plsc.load_gather(stage[b], [rows, idx])
                rr = r_chunk + j
                out_row0 = slab * 64 + (rr >> 7) * 2
                out_rows = jax.lax.broadcast(out_row0, (16,)) + row_off
                out_lanes = jax.lax.broadcast(rr & 127, (16,))
                plsc.store_scatter(out_v, [out_rows, out_lanes], vals)
                return _

            lax.fori_loop(0, _CHUNK, row_body, None, unroll=8)

            @pl.when(c + _NBUF < _N_CHUNKS)
            def _():
                start_fetch(c + _NBUF, b)

    pltpu.sync_copy(
        out_v, out_hbm.at[pl.ds(wid * _OUT_ROWS_PER_W, _OUT_ROWS_PER_W), :]
    )


def _sc_gather(x2, gflat):
    mesh = plsc.VectorSubcoreMesh(core_axis_name="c", subcore_axis_name="s")
    run = functools.partial(
        pl.kernel,
        mesh=mesh,
        compiler_params=pltpu.CompilerParams(needs_layout_passes=False),
        out_type=jax.ShapeDtypeStruct((_SC_ROWS * 16 // 128, 128), jnp.float32),
        scratch_types=[
            pltpu.VMEM((16,), jnp.float32),
            pltpu.VMEM((_CHUNK, _W), jnp.float32),
            pltpu.VMEM((_CHUNK, _W), jnp.float32),
            pltpu.VMEM((_CHUNK, _W), jnp.float32),
            pltpu.VMEM((_CHUNK, _W), jnp.float32),
            pltpu.VMEM((_CHUNK, _W), jnp.float32),
            pltpu.VMEM((_CHUNK, _W), jnp.float32),
            pltpu.VMEM((_CHUNK, _W), jnp.float32),
            pltpu.VMEM((_CHUNK, _W), jnp.float32),
            pltpu.VMEM((_CHUNK, _W), jnp.float32),
            pltpu.VMEM((_CHUNK, _W), jnp.float32),
            pltpu.VMEM((_CHUNK, _W), jnp.float32),
            pltpu.VMEM((_CHUNK, _W), jnp.float32),
            pltpu.VMEM((_CHUNK, _W), jnp.float32),
            pltpu.VMEM((_CHUNK, _W), jnp.float32),
            pltpu.VMEM((_CHUNK, _W), jnp.float32),
            pltpu.VMEM((_CHUNK, _W), jnp.float32),
            pltpu.VMEM((_OUT_ROWS_PER_W, 128), jnp.float32),
            pltpu.SemaphoreType.DMA((_NBUF,)),
        ],
    )(_sc_kernel)
    return run(x2, gflat)


def _tc_kernel(grid_ref, x_ref, out_ref):
    g = grid_ref[0:1, :]
    gi = jnp.round((g + 1.0) * (_W - 1) / 2.0).astype(jnp.int32)
    gi = jnp.clip(gi, 0, _W - 1)
    lane = jax.lax.broadcasted_iota(jnp.int32, (_W, 16), 0)
    onehot = (lane == jnp.broadcast_to(gi, (_W, 16))).astype(jnp.float32)
    yt = jax.lax.dot_general(
        onehot, x_ref[...], (((0,), (1,)), ((), ())),
        preferred_element_type=jnp.float32,
        precision=jax.lax.Precision.HIGHEST,
    )
    for rt in range(4):
        for k in range(16):
            row = (k >> 1) * 8 + rt * 2 + (k & 1)
            out_ref[row:row + 1, :] = yt[k:k + 1, rt * 128:(rt + 1) * 128]


def _tc_gather(x2, grid):
    n_tc_slabs = _N_SLABS - _SC_SLABS
    g2d = jnp.broadcast_to(grid.reshape(1, -1), (8, grid.size))
    return pl.pallas_call(
        _tc_kernel,
        grid=(n_tc_slabs,),
        in_specs=[
            pl.BlockSpec((8, 16), lambda s: (0, 0)),
            pl.BlockSpec((512, _W), lambda s: (_SC_SLABS + s, 0)),
        ],
        out_specs=pl.BlockSpec((64, 128), lambda s: (s, 0)),
        out_shape=jax.ShapeDtypeStruct((n_tc_slabs * 64, 128), jnp.float32),
    )(g2d, x2)


def kernel(x, grid):
    b, c, r, w = x.shape
    x2 = x.reshape(b * c * r, w)
    gflat = grid.reshape(grid.size)

    sc_out = _sc_gather(x2, gflat)
    tc_out = _tc_gather(x2, grid)
    out = jnp.concatenate([sc_out, tc_out], axis=0)

    out6 = (
        out.reshape(b, c, 8, 4, 2, 128)
        .transpose(0, 1, 3, 5, 2, 4)
        .reshape(b, c, r, 8, 1, 2)
    )
    return out6


# speedup vs baseline: 1.4736x; 1.0129x over previous
import functools

import jax
import jax.numpy as jnp
from jax import lax
from jax.experimental import pallas as pl
from jax.experimental.pallas import tpu as pltpu
from jax.experimental.pallas import tpu_sc as plsc

_N_ROWS = 65536
_W = 512
_N_SLABS = 128
_SC_SLABS = 96
_N_WORKERS = 32
_SC_ROWS = _SC_SLABS * 512
_ROWS_PER_W = _SC_ROWS // _N_WORKERS
_CHUNK = 16
_N_CHUNKS = _ROWS_PER_W // _CHUNK
_NBUF = 8
_OUT_ROWS_PER_W = _ROWS_PER_W * 16 // 128
_MAGIC = 12582912.0


def _sc_kernel(x_hbm, g_hbm, out_hbm, g_v, stage0_v, stage1_v, stage2_v,
               stage3_v, stage4_v, stage5_v, stage6_v, stage7_v, out_v, sems):
    stage = (stage0_v, stage1_v, stage2_v, stage3_v, stage4_v, stage5_v,
             stage6_v, stage7_v)
    wid = lax.axis_index("s") * 2 + lax.axis_index("c")
    row_base = wid * _ROWS_PER_W

    pltpu.sync_copy(g_hbm, g_v)
    g = g_v[...]
    t = (g + 1.0) * ((_W - 1) / 2.0)
    r = (t + _MAGIC) - _MAGIC
    idx = r.astype(jnp.int32)
    idx = jnp.minimum(jnp.maximum(idx, 0), _W - 1)

    k_iota = lax.iota(jnp.int32, 16)
    row_off = (k_iota >> 1) * 8 + (k_iota & 1)

    def start_fetch(c, b):
        pltpu.async_copy(
            x_hbm.at[pl.ds(row_base + c * _CHUNK, _CHUNK), :],
            stage[b],
            sems.at[b],
        )

    def wait_fetch(c, b):
        pltpu.make_async_copy(
            x_hbm.at[pl.ds(row_base + c * _CHUNK, _CHUNK), :],
            stage[b],
            sems.at[b],
        ).wait()

    for b in range(_NBUF):
        start_fetch(b, b)

    @pl.loop(0, _N_CHUNKS, step=_NBUF)
    def chunk_group(c0):
        for b in range(_NBUF):
            c = c0 + b
            wait_fetch(c, b)
            cps = 512 // _CHUNK
            slab = c // cps
            r_chunk = (c % cps) * _CHUNK

            def row_body(j, _):
                rows = jax.lax.broadcast(j, (16,))
                vals = plsc.load_gather(stage[b], [rows, idx])
                rr = r_chunk + j
                out_row0 = slab * 64 + (rr >> 7) * 2
                out_rows = jax.lax.broadcast(out_row0, (16,)) + row_off
                out_lanes = jax.lax.broadcast(rr & 127, (16,))
                plsc.store_scatter(out_v, [out_rows, out_lanes], vals)
                return _

            lax.fori_loop(0, _CHUNK, row_body, None, unroll=8)

            @pl.when(c + _NBUF < _N_CHUNKS)
            def _():
                start_fetch(c + _NBUF, b)

    pltpu.sync_copy(
        out_v, out_hbm.at[pl.ds(wid * _OUT_ROWS_PER_W, _OUT_ROWS_PER_W), :]
    )


def _sc_gather(x2, gflat):
    mesh = plsc.VectorSubcoreMesh(core_axis_name="c", subcore_axis_name="s")
    run = functools.partial(
        pl.kernel,
        mesh=mesh,
        compiler_params=pltpu.CompilerParams(needs_layout_passes=False),
        out_type=jax.ShapeDtypeStruct((_SC_ROWS * 16 // 128, 128), jnp.float32),
        scratch_types=[
            pltpu.VMEM((16,), jnp.float32),
            pltpu.VMEM((_CHUNK, _W), jnp.float32),
            pltpu.VMEM((_CHUNK, _W), jnp.float32),
            pltpu.VMEM((_CHUNK, _W), jnp.float32),
            pltpu.VMEM((_CHUNK, _W), jnp.float32),
            pltpu.VMEM((_CHUNK, _W), jnp.float32),
            pltpu.VMEM((_CHUNK, _W), jnp.float32),
            pltpu.VMEM((_CHUNK, _W), jnp.float32),
            pltpu.VMEM((_CHUNK, _W), jnp.float32),
            pltpu.VMEM((_OUT_ROWS_PER_W, 128), jnp.float32),
            pltpu.SemaphoreType.DMA((_NBUF,)),
        ],
    )(_sc_kernel)
    return run(x2, gflat)


def _tc_kernel(grid_ref, x_ref, out_ref):
    g = grid_ref[0:1, :]
    gi = jnp.round((g + 1.0) * (_W - 1) / 2.0).astype(jnp.int32)
    gi = jnp.clip(gi, 0, _W - 1)
    lane = jax.lax.broadcasted_iota(jnp.int32, (_W, 16), 0)
    onehot = (lane == jnp.broadcast_to(gi, (_W, 16))).astype(jnp.float32)
    yt = jax.lax.dot_general(
        onehot, x_ref[...], (((0,), (1,)), ((), ())),
        preferred_element_type=jnp.float32,
        precision=jax.lax.Precision.HIGHEST,
    )
    for rt in range(4):
        for k in range(16):
            row = (k >> 1) * 8 + rt * 2 + (k & 1)
            out_ref[row:row + 1, :] = yt[k:k + 1, rt * 128:(rt + 1) * 128]


def _tc_gather(x2, grid):
    n_tc_slabs = _N_SLABS - _SC_SLABS
    g2d = jnp.broadcast_to(grid.reshape(1, -1), (8, grid.size))
    return pl.pallas_call(
        _tc_kernel,
        grid=(n_tc_slabs,),
        in_specs=[
            pl.BlockSpec((8, 16), lambda s: (0, 0)),
            pl.BlockSpec((512, _W), lambda s: (_SC_SLABS + s, 0)),
        ],
        out_specs=pl.BlockSpec((64, 128), lambda s: (s, 0)),
        out_shape=jax.ShapeDtypeStruct((n_tc_slabs * 64, 128), jnp.float32),
    )(g2d, x2)


def kernel(x, grid):
    b, c, r, w = x.shape
    x2 = x.reshape(b * c * r, w)
    gflat = grid.reshape(grid.size)

    sc_out = _sc_gather(x2, gflat)
    tc_out = _tc_gather(x2, grid)
    out = jnp.concatenate([sc_out, tc_out], axis=0)

    out6 = (
        out.reshape(b, c, 8, 4, 2, 128)
        .transpose(0, 1, 3, 5, 2, 4)
        .reshape(b, c, r, 8, 1, 2)
    )
    return out6
